# Initial kernel scaffold; baseline (speedup 1.0000x reference)
import jax
import jax.numpy as jnp
from jax.experimental import pallas as pl


def kernel(table, query_indices, seq_indices):
    raise NotImplementedError("write your pallas kernel here")


# speedup vs baseline: 1.4327x; 1.4327x over previous
import functools

import jax
import jax.numpy as jnp
from jax import lax
from jax.experimental import pallas as pl
from jax.experimental.pallas import tpu as pltpu
from jax.experimental.pallas import tpu_sc as plsc

NC, NS = 2, 16
NW = NC * NS
D = 16
N_FIELDS = 226
NCHUNK = 16


@functools.partial(jax.jit, static_argnums=())
def _sc_gather(table, idx_flat):
    n_rows = idx_flat.shape[0]
    per_w = n_rows // NW
    chunk = per_w // NCHUNK

    mesh = plsc.VectorSubcoreMesh(core_axis_name="c", subcore_axis_name="s")

    @functools.partial(
        pl.kernel,
        mesh=mesh,
        out_type=jax.ShapeDtypeStruct((n_rows, D), jnp.float32),
        scratch_types=[
            pltpu.VMEM((per_w,), jnp.int32),
            pltpu.VMEM((chunk, D), jnp.float32),
            pltpu.SemaphoreType.DMA,
        ],
        compiler_params=pltpu.CompilerParams(use_tc_tiling_on_sc=False),
    )
    def gather_kernel(table_hbm, idx_hbm, out_hbm, idx_v, rows_v, sem):
        wid = lax.axis_index("s") * NC + lax.axis_index("c")
        base = wid * per_w
        pltpu.sync_copy(idx_hbm.at[pl.ds(base, per_w)], idx_v)
        for i in range(NCHUNK):
            pltpu.async_copy(
                table_hbm.at[idx_v.at[pl.ds(i * chunk, chunk)]], rows_v, sem
            ).wait()
            pltpu.sync_copy(rows_v, out_hbm.at[pl.ds(base + i * chunk, chunk)])

    return gather_kernel(table, idx_flat)


def kernel(table, query_indices, seq_indices):
    b = query_indices.shape[0]
    idx_flat = jnp.concatenate([query_indices, seq_indices], axis=1).reshape(-1)
    out = _sc_gather(table, idx_flat)
    return out.reshape(b, -1)


# speedup vs baseline: 1.4584x; 1.0179x over previous
import functools

import jax
import jax.numpy as jnp
from jax import lax
from jax.experimental import pallas as pl
from jax.experimental.pallas import tpu as pltpu
from jax.experimental.pallas import tpu_sc as plsc

NC, NS = 2, 16
NW = NC * NS
D = 16
N_FIELDS = 226
NCHUNK = 16


@functools.partial(jax.jit, static_argnums=())
def _sc_gather(table, idx_flat):
    n_rows = idx_flat.shape[0]
    per_w = n_rows // NW
    chunk = per_w // NCHUNK

    mesh = plsc.VectorSubcoreMesh(core_axis_name="c", subcore_axis_name="s")

    nbuf = 2

    @functools.partial(
        pl.kernel,
        mesh=mesh,
        out_type=jax.ShapeDtypeStruct((n_rows, D), jnp.float32),
        scratch_types=[
            pltpu.VMEM((per_w,), jnp.int32),
            [pltpu.VMEM((chunk, D), jnp.float32) for _ in range(nbuf)],
            [pltpu.SemaphoreType.DMA for _ in range(nbuf)],
            [pltpu.SemaphoreType.DMA for _ in range(nbuf)],
        ],
        compiler_params=pltpu.CompilerParams(use_tc_tiling_on_sc=False),
    )
    def gather_kernel(table_hbm, idx_hbm, out_hbm, idx_v, rows, gsem, osem):
        wid = lax.axis_index("s") * NC + lax.axis_index("c")
        base = wid * per_w
        pltpu.sync_copy(idx_hbm.at[pl.ds(base, per_w)], idx_v)

        def start_gather(i):
            b = i % nbuf
            return pltpu.async_copy(
                table_hbm.at[idx_v.at[pl.ds(i * chunk, chunk)]], rows[b], gsem[b]
            )

        gathers = {0: start_gather(0)}
        ocopies = {}
        for i in range(NCHUNK):
            b = i % nbuf
            if i + 1 < NCHUNK:
                if i + 1 >= nbuf:
                    ocopies.pop(i + 1 - nbuf).wait()
                gathers[i + 1] = start_gather(i + 1)
            gathers.pop(i).wait()
            ocopies[i] = pltpu.async_copy(
                rows[b], out_hbm.at[pl.ds(base + i * chunk, chunk)], osem[b]
            )
        for i in sorted(ocopies):
            ocopies.pop(i).wait()

    return gather_kernel(table, idx_flat)


def kernel(table, query_indices, seq_indices):
    b = query_indices.shape[0]
    idx_flat = jnp.concatenate([query_indices, seq_indices], axis=1).reshape(-1)
    out = _sc_gather(table, idx_flat)
    return out.reshape(b, -1)


# speedup vs baseline: 1.4613x; 1.0020x over previous
import functools

import jax
import jax.numpy as jnp
from jax import lax
from jax.experimental import pallas as pl
from jax.experimental.pallas import tpu as pltpu
from jax.experimental.pallas import tpu_sc as plsc

NC, NS = 2, 16
NW = NC * NS
D = 16
N_FIELDS = 226
NCHUNK = 32


@functools.partial(jax.jit, static_argnums=())
def _sc_gather(table, idx_flat):
    n_rows = idx_flat.shape[0]
    per_w = n_rows // NW
    chunk = per_w // NCHUNK

    mesh = plsc.VectorSubcoreMesh(core_axis_name="c", subcore_axis_name="s")

    nbuf = 4

    @functools.partial(
        pl.kernel,
        mesh=mesh,
        out_type=jax.ShapeDtypeStruct((n_rows, D), jnp.float32),
        scratch_types=[
            pltpu.VMEM((per_w,), jnp.int32),
            [pltpu.VMEM((chunk, D), jnp.float32) for _ in range(nbuf)],
            [pltpu.SemaphoreType.DMA for _ in range(nbuf)],
            [pltpu.SemaphoreType.DMA for _ in range(nbuf)],
        ],
        compiler_params=pltpu.CompilerParams(use_tc_tiling_on_sc=False),
    )
    def gather_kernel(table_hbm, idx_hbm, out_hbm, idx_v, rows, gsem, osem):
        wid = lax.axis_index("s") * NC + lax.axis_index("c")
        base = wid * per_w
        pltpu.sync_copy(idx_hbm.at[pl.ds(base, per_w)], idx_v)

        def start_gather(i):
            b = i % nbuf
            return pltpu.async_copy(
                table_hbm.at[idx_v.at[pl.ds(i * chunk, chunk)]], rows[b], gsem[b]
            )

        gathers = {}
        ocopies = {}
        next_g = 0
        for i in range(NCHUNK):
            while next_g < min(NCHUNK, i + nbuf):
                if next_g >= nbuf:
                    ocopies.pop(next_g - nbuf).wait()
                gathers[next_g] = start_gather(next_g)
                next_g += 1
            b = i % nbuf
            gathers.pop(i).wait()
            ocopies[i] = pltpu.async_copy(
                rows[b], out_hbm.at[pl.ds(base + i * chunk, chunk)], osem[b]
            )
        for i in sorted(ocopies):
            ocopies.pop(i).wait()

    return gather_kernel(table, idx_flat)


def kernel(table, query_indices, seq_indices):
    b = query_indices.shape[0]
    idx_flat = jnp.concatenate([query_indices, seq_indices], axis=1).reshape(-1)
    out = _sc_gather(table, idx_flat)
    return out.reshape(b, -1)


# speedup vs baseline: 2.4166x; 1.6537x over previous
import functools

import jax
import jax.numpy as jnp
from jax import lax
from jax.experimental import pallas as pl
from jax.experimental.pallas import tpu as pltpu
from jax.experimental.pallas import tpu_sc as plsc

NC, NS = 2, 16
NW = NC * NS
D = 16
NCHUNK = 32

V = 1000000
LANE = 128
NCOL = V // LANE
REM = V - NCOL * LANE
GCOL = 5
CPW = 245


def _sc_linearize(table_t):
    mesh = plsc.VectorSubcoreMesh(core_axis_name="c", subcore_axis_name="s")
    blk = GCOL * LANE

    @functools.partial(
        pl.kernel,
        mesh=mesh,
        out_type=jax.ShapeDtypeStruct((V * D,), jnp.float32),
        scratch_types=[
            pltpu.VMEM((D, blk), jnp.float32),
            pltpu.VMEM((blk * D,), jnp.float32),
            pltpu.VMEM((D, REM), jnp.float32),
            pltpu.VMEM((REM * D,), jnp.float32),
        ],
        compiler_params=pltpu.CompilerParams(
            use_tc_tiling_on_sc=True, needs_layout_passes=False
        ),
    )
    def linearize_kernel(tab_hbm, out_hbm, in_v, out_v, rin_v, rout_v):
        wid = lax.axis_index("s") * NC + lax.axis_index("c")
        start = lax.min(wid * CPW, NCOL - CPW)
        base16 = lax.broadcasted_iota(jnp.int32, (16,), 0) * D

        def do_block(c0, src, dst, width):
            pltpu.sync_copy(tab_hbm.at[:, pl.ds(c0 * LANE, width)], src)

            def body(j, _):
                for f in range(D):
                    v = src[f, pl.ds(j * D, D)]
                    idx = base16 + (j * (D * D) + f)
                    plsc.store_scatter(dst, [idx], v)
                return 0

            lax.fori_loop(0, width // D, body, 0)
            pltpu.sync_copy(dst, out_hbm.at[pl.ds(c0 * LANE * D, width * D)])

        def blocks(b, _):
            do_block(start + b * GCOL, in_v, out_v, blk)
            return 0

        lax.fori_loop(0, CPW // GCOL, blocks, 0)

        @pl.when(wid == 0)
        def _():
            do_block(NCOL, rin_v, rout_v, REM)

    return linearize_kernel(table_t)


def _sc_gather(table, idx_flat):
    n_rows = idx_flat.shape[0]
    per_w = n_rows // NW
    chunk = per_w // NCHUNK

    mesh = plsc.VectorSubcoreMesh(core_axis_name="c", subcore_axis_name="s")
    nbuf = 4

    @functools.partial(
        pl.kernel,
        mesh=mesh,
        out_type=jax.ShapeDtypeStruct((n_rows, D), jnp.float32),
        scratch_types=[
            pltpu.VMEM((per_w,), jnp.int32),
            [pltpu.VMEM((chunk, D), jnp.float32) for _ in range(nbuf)],
            [pltpu.SemaphoreType.DMA for _ in range(nbuf)],
            [pltpu.SemaphoreType.DMA for _ in range(nbuf)],
        ],
        compiler_params=pltpu.CompilerParams(use_tc_tiling_on_sc=False),
    )
    def gather_kernel(table_hbm, idx_hbm, out_hbm, idx_v, rows, gsem, osem):
        wid = lax.axis_index("s") * NC + lax.axis_index("c")
        base = wid * per_w
        pltpu.sync_copy(idx_hbm.at[pl.ds(base, per_w)], idx_v)

        def start_gather(i):
            b = i % nbuf
            return pltpu.async_copy(
                table_hbm.at[idx_v.at[pl.ds(i * chunk, chunk)]], rows[b], gsem[b]
            )

        gathers = {}
        ocopies = {}
        next_g = 0
        for i in range(NCHUNK):
            while next_g < min(NCHUNK, i + nbuf):
                if next_g >= nbuf:
                    ocopies.pop(next_g - nbuf).wait()
                gathers[next_g] = start_gather(next_g)
                next_g += 1
            b = i % nbuf
            gathers.pop(i).wait()
            ocopies[i] = pltpu.async_copy(
                rows[b], out_hbm.at[pl.ds(base + i * chunk, chunk)], osem[b]
            )
        for i in sorted(ocopies):
            ocopies.pop(i).wait()

    return gather_kernel(table, idx_flat)


def kernel(table, query_indices, seq_indices):
    b = query_indices.shape[0]
    tbl_flat = _sc_linearize(table.T)
    tbl = tbl_flat.reshape(V, D)
    idx_flat = jnp.concatenate([query_indices, seq_indices], axis=1).reshape(-1)
    out = _sc_gather(tbl, idx_flat)
    return out.reshape(b, -1)


# speedup vs baseline: 3.0271x; 1.2526x over previous
import functools

import jax
import jax.numpy as jnp
from jax import lax
from jax.experimental import pallas as pl
from jax.experimental.pallas import tpu as pltpu
from jax.experimental.pallas import tpu_sc as plsc

NC, NS = 2, 16
NW = NC * NS
D = 16
NCHUNK = 32

V = 1000000
LANE = 128
NCOL = V // LANE
REM = V - NCOL * LANE
GCOL = 5
CPW = 250


def _sc_linearize(table_t):
    mesh = plsc.VectorSubcoreMesh(core_axis_name="c", subcore_axis_name="s")
    blk = GCOL * LANE

    nblk = CPW // GCOL

    @functools.partial(
        pl.kernel,
        mesh=mesh,
        out_type=jax.ShapeDtypeStruct((V * D,), jnp.float32),
        scratch_types=[
            [pltpu.VMEM((D, blk), jnp.float32) for _ in range(2)],
            [pltpu.VMEM((blk * D,), jnp.float32) for _ in range(2)],
            pltpu.VMEM((D, REM), jnp.float32),
            pltpu.VMEM((REM * D,), jnp.float32),
            [pltpu.SemaphoreType.DMA for _ in range(2)],
            [pltpu.SemaphoreType.DMA for _ in range(2)],
        ],
        compiler_params=pltpu.CompilerParams(
            use_tc_tiling_on_sc=True, needs_layout_passes=False
        ),
    )
    def linearize_kernel(tab_hbm, out_hbm, in_v, out_v, rin_v, rout_v, isem, osem):
        wid = lax.axis_index("s") * NC + lax.axis_index("c")
        start = lax.min(wid * CPW, NCOL - CPW)
        base16 = lax.broadcasted_iota(jnp.int32, (16,), 0) * D

        def transpose_block(src, dst, width):
            def body(j, idx0):
                for u in range(2):
                    idx_base = idx0 + u * (D * D)
                    for f in range(D):
                        v = src[f, pl.ds((2 * j + u) * D, D)]
                        plsc.store_scatter(dst, [idx_base + f], v)
                return idx0 + 2 * D * D

            lax.fori_loop(0, width // (2 * D), body, base16)

        def start_in(b, u):
            c0 = start + b * GCOL
            pltpu.async_copy(tab_hbm.at[:, pl.ds(c0 * LANE, blk)], in_v[u], isem[u])

        def start_out(b, u):
            c0 = start + b * GCOL
            pltpu.async_copy(
                out_v[u], out_hbm.at[pl.ds(c0 * LANE * D, blk * D)], osem[u]
            )

        def wait_in(u):
            pltpu.make_async_copy(
                tab_hbm.at[:, pl.ds(0, blk)], in_v[u], isem[u]
            ).wait()

        def wait_out(u):
            pltpu.make_async_copy(
                out_v[u], out_hbm.at[pl.ds(0, blk * D)], osem[u]
            ).wait()

        start_in(0, 0)
        start_in(1, 1)

        def body(p, _):
            for u in range(2):
                b = 2 * p + u
                wait_in(u)

                @pl.when(b >= 2)
                def _():
                    wait_out(u)

                transpose_block(in_v[u], out_v[u], blk)
                start_out(b, u)

                @pl.when(b + 2 < nblk)
                def _():
                    start_in(b + 2, u)

            return 0

        lax.fori_loop(0, nblk // 2, body, 0)
        wait_out(0)
        wait_out(1)

        @pl.when(wid == 0)
        def _():
            pltpu.sync_copy(tab_hbm.at[:, pl.ds(NCOL * LANE, REM)], rin_v)
            transpose_block(rin_v, rout_v, REM)
            pltpu.sync_copy(rout_v, out_hbm.at[pl.ds(NCOL * LANE * D, REM * D)])

    return linearize_kernel(table_t)


def _sc_gather(table, idx_flat):
    n_rows = idx_flat.shape[0]
    per_w = n_rows // NW
    chunk = per_w // NCHUNK

    mesh = plsc.VectorSubcoreMesh(core_axis_name="c", subcore_axis_name="s")
    nbuf = 4

    @functools.partial(
        pl.kernel,
        mesh=mesh,
        out_type=jax.ShapeDtypeStruct((n_rows, D), jnp.float32),
        scratch_types=[
            pltpu.VMEM((per_w,), jnp.int32),
            [pltpu.VMEM((chunk, D), jnp.float32) for _ in range(nbuf)],
            [pltpu.SemaphoreType.DMA for _ in range(nbuf)],
            [pltpu.SemaphoreType.DMA for _ in range(nbuf)],
        ],
        compiler_params=pltpu.CompilerParams(use_tc_tiling_on_sc=False),
    )
    def gather_kernel(table_hbm, idx_hbm, out_hbm, idx_v, rows, gsem, osem):
        wid = lax.axis_index("s") * NC + lax.axis_index("c")
        base = wid * per_w
        pltpu.sync_copy(idx_hbm.at[pl.ds(base, per_w)], idx_v)

        def start_gather(i):
            b = i % nbuf
            return pltpu.async_copy(
                table_hbm.at[idx_v.at[pl.ds(i * chunk, chunk)]], rows[b], gsem[b]
            )

        gathers = {}
        ocopies = {}
        next_g = 0
        for i in range(NCHUNK):
            while next_g < min(NCHUNK, i + nbuf):
                if next_g >= nbuf:
                    ocopies.pop(next_g - nbuf).wait()
                gathers[next_g] = start_gather(next_g)
                next_g += 1
            b = i % nbuf
            gathers.pop(i).wait()
            ocopies[i] = pltpu.async_copy(
                rows[b], out_hbm.at[pl.ds(base + i * chunk, chunk)], osem[b]
            )
        for i in sorted(ocopies):
            ocopies.pop(i).wait()

    return gather_kernel(table, idx_flat)


def kernel(table, query_indices, seq_indices):
    b = query_indices.shape[0]
    tbl_flat = _sc_linearize(table.T)
    tbl = tbl_flat.reshape(V, D)
    idx_flat = jnp.concatenate([query_indices, seq_indices], axis=1).reshape(-1)
    out = _sc_gather(tbl, idx_flat)
    return out.reshape(b, -1)
